# initial kernel scaffold (unmeasured)
import jax
import jax.numpy as jnp
from jax import lax
from jax.experimental import pallas as pl
from jax.experimental.pallas import tpu as pltpu

N_DEV = 32


def kernel(q, k, v):
    S_blk, D = q.shape
    TQ = 512
    NT = S_blk // TQ
    scale = 1.0 / (D**0.5)

    q = q.astype(jnp.bfloat16)
    k = k.astype(jnp.bfloat16)
    v = v.astype(jnp.bfloat16)

    def body(q_ref, k_ref, v_ref, out_ref, comm_ref, acc_ref, m_ref, l_ref,
             send_sem, recv_sem):
        my = lax.axis_index("i")
        left = lax.rem(my - 1 + N_DEV, N_DEV)
        right = lax.rem(my + 1, N_DEV)

        barrier = pltpu.get_barrier_semaphore()
        for nbr in (left, right):
            pl.semaphore_signal(
                barrier, inc=1,
                device_id=(nbr,), device_id_type=pl.DeviceIdType.MESH,
            )
        pl.semaphore_wait(barrier, 2)

        comm_ref[0, 0] = k_ref[...]
        comm_ref[0, 1] = v_ref[...]
        acc_ref[...] = jnp.zeros((S_blk, D), jnp.float32)
        m_ref[...] = jnp.full((S_blk, 1), -1e30, jnp.float32)
        l_ref[...] = jnp.zeros((S_blk, 1), jnp.float32)

        def hop(h, carry):
            slot = lax.rem(h, 2)
            nxt = 1 - slot
            rdma = pltpu.make_async_remote_copy(
                src_ref=comm_ref.at[slot],
                dst_ref=comm_ref.at[nxt],
                send_sem=send_sem.at[slot],
                recv_sem=recv_sem.at[nxt],
                device_id=(right,),
                device_id_type=pl.DeviceIdType.MESH,
            )

            @pl.when(h < N_DEV - 1)
            def _():
                rdma.start()
                rdma.wait()

            k_blk = comm_ref[slot, 0]
            v_blk = comm_ref[slot, 1]
            for t in range(NT):
                r = pl.ds(t * TQ, TQ)
                q_t = q_ref[r, :]
                s = lax.dot_general(
                    q_t, k_blk, (((1,), (1,)), ((), ())),
                    preferred_element_type=jnp.float32,
                ) * scale
                m_prev = m_ref[r, :]
                l_prev = l_ref[r, :]
                m_cur = jnp.maximum(m_prev, jnp.max(s, axis=1, keepdims=True))
                alpha = jnp.exp(m_prev - m_cur)
                p = jnp.exp(s - m_cur)
                l_ref[r, :] = l_prev * alpha + jnp.sum(p, axis=1, keepdims=True)
                pv = lax.dot_general(
                    p.astype(jnp.bfloat16), v_blk, (((1,), (0,)), ((), ())),
                    preferred_element_type=jnp.float32,
                )
                acc_ref[r, :] = acc_ref[r, :] * alpha + pv
                m_ref[r, :] = m_cur
            return carry

        lax.fori_loop(0, N_DEV, hop, 0)
        out_ref[...] = acc_ref[...] / l_ref[...]

    return pl.pallas_call(
        body,
        out_shape=jax.ShapeDtypeStruct((S_blk, D), jnp.float32),
        in_specs=[
            pl.BlockSpec(memory_space=pltpu.VMEM),
            pl.BlockSpec(memory_space=pltpu.VMEM),
            pl.BlockSpec(memory_space=pltpu.VMEM),
        ],
        out_specs=pl.BlockSpec(memory_space=pltpu.VMEM),
        scratch_shapes=[
            pltpu.VMEM((2, 2, S_blk, D), jnp.bfloat16),
            pltpu.VMEM((S_blk, D), jnp.float32),
            pltpu.VMEM((S_blk, 1), jnp.float32),
            pltpu.VMEM((S_blk, 1), jnp.float32),
            pltpu.SemaphoreType.DMA((2,)),
            pltpu.SemaphoreType.DMA((2,)),
        ],
        compiler_params=pltpu.CompilerParams(collective_id=0),
    )(q, k, v)


# baseline (device time: 5861236 ns/iter reference)
import jax
import jax.numpy as jnp
from jax import lax
from jax.experimental import pallas as pl
from jax.experimental.pallas import tpu as pltpu

N_DEV = 32


def kernel(q, k, v):
    S_blk, D = q.shape
    TQ = min(128, S_blk)
    NT = S_blk // TQ
    scale = 1.0 / (D**0.5)

    q = q.astype(jnp.bfloat16)
    k = k.astype(jnp.bfloat16)
    v = v.astype(jnp.bfloat16)

    def body(q_ref, k_hbm, v_hbm, out_ref, comm_ref, m_ref, l_ref,
             send_sem, recv_sem, local_sem):
        my = lax.axis_index("i")
        left = lax.rem(my - 1 + N_DEV, N_DEV)
        right = lax.rem(my + 1, N_DEV)

        barrier = pltpu.get_barrier_semaphore()
        for nbr in (left, right):
            pl.semaphore_signal(
                barrier, inc=1,
                device_id=(nbr,), device_id_type=pl.DeviceIdType.MESH,
            )
        pl.semaphore_wait(barrier, 2)

        cp_k = pltpu.make_async_copy(k_hbm, comm_ref.at[0, 0], local_sem.at[0])
        cp_v = pltpu.make_async_copy(v_hbm, comm_ref.at[0, 1], local_sem.at[1])
        cp_k.start()
        cp_v.start()
        out_ref[...] = jnp.zeros((S_blk, D), jnp.float32)
        m_ref[...] = jnp.full((S_blk, 1), -1e30, jnp.float32)
        l_ref[...] = jnp.zeros((S_blk, 1), jnp.float32)
        cp_k.wait()
        cp_v.wait()

        def hop(h, carry):
            slot = lax.rem(h, 2)
            nxt = 1 - slot
            rdma = pltpu.make_async_remote_copy(
                src_ref=comm_ref.at[slot],
                dst_ref=comm_ref.at[nxt],
                send_sem=send_sem.at[slot],
                recv_sem=recv_sem.at[nxt],
                device_id=(right,),
                device_id_type=pl.DeviceIdType.MESH,
            )

            @pl.when(h < N_DEV - 1)
            def _():
                rdma.start()
                rdma.wait()

            k_blk = comm_ref[slot, 0]
            v_blk = comm_ref[slot, 1]

            def tile(t, c):
                r = pl.ds(t * TQ, TQ)
                q_t = q_ref[r, :]
                s = lax.dot_general(
                    q_t, k_blk, (((1,), (1,)), ((), ())),
                    preferred_element_type=jnp.float32,
                ) * scale
                m_prev = m_ref[r, :]
                l_prev = l_ref[r, :]
                m_cur = jnp.maximum(m_prev, jnp.max(s, axis=1, keepdims=True))
                alpha = jnp.exp(m_prev - m_cur)
                p = jnp.exp(s - m_cur)
                l_ref[r, :] = l_prev * alpha + jnp.sum(p, axis=1, keepdims=True)
                pv = lax.dot_general(
                    p.astype(jnp.bfloat16), v_blk, (((1,), (0,)), ((), ())),
                    preferred_element_type=jnp.float32,
                )
                out_ref[r, :] = out_ref[r, :] * alpha + pv
                m_ref[r, :] = m_cur
                return c

            lax.fori_loop(0, NT, tile, 0)
            return carry

        lax.fori_loop(0, N_DEV, hop, 0)
        out_ref[...] = out_ref[...] / l_ref[...]

    return pl.pallas_call(
        body,
        out_shape=jax.ShapeDtypeStruct((S_blk, D), jnp.float32),
        in_specs=[
            pl.BlockSpec(memory_space=pltpu.VMEM),
            pl.BlockSpec(memory_space=pltpu.MemorySpace.HBM),
            pl.BlockSpec(memory_space=pltpu.MemorySpace.HBM),
        ],
        out_specs=pl.BlockSpec(memory_space=pltpu.VMEM),
        scratch_shapes=[
            pltpu.VMEM((2, 2, S_blk, D), jnp.bfloat16),
            pltpu.VMEM((S_blk, 1), jnp.float32),
            pltpu.VMEM((S_blk, 1), jnp.float32),
            pltpu.SemaphoreType.DMA((2,)),
            pltpu.SemaphoreType.DMA((2,)),
            pltpu.SemaphoreType.DMA((2,)),
        ],
        compiler_params=pltpu.CompilerParams(
            collective_id=0,
            vmem_limit_bytes=63 * 1024 * 1024,
        ),
    )(q, k, v)


# device time: 3018192 ns/iter; 1.9420x vs baseline; 1.9420x over previous
import jax
import jax.numpy as jnp
from jax import lax
from jax.experimental import pallas as pl
from jax.experimental.pallas import tpu as pltpu

N_DEV = 32


def kernel(q, k, v):
    S_blk, D = q.shape
    TQ = min(128, S_blk)
    NT = S_blk // TQ
    scale = 1.0 / (D**0.5)

    q = q.astype(jnp.bfloat16)
    k = k.astype(jnp.bfloat16)
    v = v.astype(jnp.bfloat16)

    def body(q_ref, k_hbm, v_hbm, out_ref, comm_ref, m_ref, l_ref,
             send_sem, recv_sem, local_sem):
        my = lax.axis_index("i")
        left = lax.rem(my - 1 + N_DEV, N_DEV)
        right = lax.rem(my + 1, N_DEV)

        barrier = pltpu.get_barrier_semaphore()
        for nbr in (left, right):
            pl.semaphore_signal(
                barrier, inc=1,
                device_id=(nbr,), device_id_type=pl.DeviceIdType.MESH,
            )
        pl.semaphore_wait(barrier, 2)

        cp_k = pltpu.make_async_copy(k_hbm, comm_ref.at[0, 0], local_sem.at[0])
        cp_v = pltpu.make_async_copy(v_hbm, comm_ref.at[0, 1], local_sem.at[1])
        cp_k.start()
        cp_v.start()
        out_ref[...] = jnp.zeros((S_blk, D), jnp.float32)
        m_ref[...] = jnp.full((S_blk, 1), -1e30, jnp.float32)
        l_ref[...] = jnp.zeros((S_blk, 1), jnp.float32)
        cp_k.wait()
        cp_v.wait()

        def hop(h, carry):
            slot = lax.rem(h, 2)
            nxt = 1 - slot
            rdma = pltpu.make_async_remote_copy(
                src_ref=comm_ref.at[slot],
                dst_ref=comm_ref.at[nxt],
                send_sem=send_sem.at[slot],
                recv_sem=recv_sem.at[nxt],
                device_id=(right,),
                device_id_type=pl.DeviceIdType.MESH,
            )

            @pl.when(h < N_DEV - 1)
            def _():
                rdma.start()

            k_blk = comm_ref[slot, 0]
            v_blk = comm_ref[slot, 1]

            def tile(t, c):
                r = pl.ds(t * TQ, TQ)
                q_t = q_ref[r, :]
                s = lax.dot_general(
                    q_t, k_blk, (((1,), (1,)), ((), ())),
                    preferred_element_type=jnp.float32,
                ) * scale
                m_prev = m_ref[r, :]
                l_prev = l_ref[r, :]
                m_cur = jnp.maximum(m_prev, jnp.max(s, axis=1, keepdims=True))
                alpha = jnp.exp(m_prev - m_cur)
                p = jnp.exp(s - m_cur)
                l_ref[r, :] = l_prev * alpha + jnp.sum(p, axis=1, keepdims=True)
                pv = lax.dot_general(
                    p.astype(jnp.bfloat16), v_blk, (((1,), (0,)), ((), ())),
                    preferred_element_type=jnp.float32,
                )
                out_ref[r, :] = out_ref[r, :] * alpha + pv
                m_ref[r, :] = m_cur
                return c

            lax.fori_loop(0, NT, tile, 0)

            @pl.when(h < N_DEV - 1)
            def _():
                rdma.wait()

            return carry

        lax.fori_loop(0, N_DEV, hop, 0)
        out_ref[...] = out_ref[...] / l_ref[...]

    return pl.pallas_call(
        body,
        out_shape=jax.ShapeDtypeStruct((S_blk, D), jnp.float32),
        in_specs=[
            pl.BlockSpec(memory_space=pltpu.VMEM),
            pl.BlockSpec(memory_space=pltpu.MemorySpace.HBM),
            pl.BlockSpec(memory_space=pltpu.MemorySpace.HBM),
        ],
        out_specs=pl.BlockSpec(memory_space=pltpu.VMEM),
        scratch_shapes=[
            pltpu.VMEM((2, 2, S_blk, D), jnp.bfloat16),
            pltpu.VMEM((S_blk, 1), jnp.float32),
            pltpu.VMEM((S_blk, 1), jnp.float32),
            pltpu.SemaphoreType.DMA((2,)),
            pltpu.SemaphoreType.DMA((2,)),
            pltpu.SemaphoreType.DMA((2,)),
        ],
        compiler_params=pltpu.CompilerParams(
            collective_id=0,
            vmem_limit_bytes=63 * 1024 * 1024,
        ),
    )(q, k, v)


# device time: 2934976 ns/iter; 1.9970x vs baseline; 1.0284x over previous
import jax
import jax.numpy as jnp
from jax import lax
from jax.experimental import pallas as pl
from jax.experimental.pallas import tpu as pltpu

N_DEV = 32


def kernel(q, k, v):
    S_blk, D = q.shape
    TQ = min(256, S_blk)
    NT = S_blk // TQ
    scale = 1.0 / (D**0.5)

    q = q.astype(jnp.bfloat16)
    k = k.astype(jnp.bfloat16)
    v = v.astype(jnp.bfloat16)

    def body(q_ref, k_hbm, v_hbm, out_ref, comm_ref, m_ref, l_ref,
             send_sem, recv_sem, local_sem):
        my = lax.axis_index("i")
        left = lax.rem(my - 1 + N_DEV, N_DEV)
        right = lax.rem(my + 1, N_DEV)

        barrier = pltpu.get_barrier_semaphore()
        for nbr in (left, right):
            pl.semaphore_signal(
                barrier, inc=1,
                device_id=(nbr,), device_id_type=pl.DeviceIdType.MESH,
            )
        pl.semaphore_wait(barrier, 2)

        cp_k = pltpu.make_async_copy(k_hbm, comm_ref.at[0, 0], local_sem.at[0])
        cp_v = pltpu.make_async_copy(v_hbm, comm_ref.at[0, 1], local_sem.at[1])
        cp_k.start()
        cp_v.start()
        out_ref[...] = jnp.zeros((S_blk, D), jnp.float32)
        m_ref[...] = jnp.full((S_blk, 1), -1e30, jnp.float32)
        l_ref[...] = jnp.zeros((S_blk, 1), jnp.float32)
        cp_k.wait()
        cp_v.wait()

        def hop(h, carry):
            slot = lax.rem(h, 2)
            nxt = 1 - slot
            rdma = pltpu.make_async_remote_copy(
                src_ref=comm_ref.at[slot],
                dst_ref=comm_ref.at[nxt],
                send_sem=send_sem.at[slot],
                recv_sem=recv_sem.at[nxt],
                device_id=(right,),
                device_id_type=pl.DeviceIdType.MESH,
            )

            @pl.when(h < N_DEV - 1)
            def _():
                rdma.start()

            k_blk = comm_ref[slot, 0]
            v_blk = comm_ref[slot, 1]

            def tile(t, c):
                r = pl.ds(t * TQ, TQ)
                q_t = q_ref[r, :]
                s = lax.dot_general(
                    q_t, k_blk, (((1,), (1,)), ((), ())),
                    preferred_element_type=jnp.float32,
                ) * scale
                m_prev = m_ref[r, :]
                l_prev = l_ref[r, :]
                m_cur = jnp.maximum(m_prev, jnp.max(s, axis=1, keepdims=True))
                alpha = jnp.exp(m_prev - m_cur)
                p = jnp.exp(s - m_cur)
                l_ref[r, :] = l_prev * alpha + jnp.sum(p, axis=1, keepdims=True)
                pv = lax.dot_general(
                    p.astype(jnp.bfloat16), v_blk, (((1,), (0,)), ((), ())),
                    preferred_element_type=jnp.float32,
                )
                out_ref[r, :] = out_ref[r, :] * alpha + pv
                m_ref[r, :] = m_cur
                return c

            lax.fori_loop(0, NT, tile, 0)

            @pl.when(h < N_DEV - 1)
            def _():
                rdma.wait()

            return carry

        lax.fori_loop(0, N_DEV, hop, 0)
        out_ref[...] = out_ref[...] / l_ref[...]

    return pl.pallas_call(
        body,
        out_shape=jax.ShapeDtypeStruct((S_blk, D), jnp.float32),
        in_specs=[
            pl.BlockSpec(memory_space=pltpu.VMEM),
            pl.BlockSpec(memory_space=pltpu.MemorySpace.HBM),
            pl.BlockSpec(memory_space=pltpu.MemorySpace.HBM),
        ],
        out_specs=pl.BlockSpec(memory_space=pltpu.VMEM),
        scratch_shapes=[
            pltpu.VMEM((2, 2, S_blk, D), jnp.bfloat16),
            pltpu.VMEM((S_blk, 1), jnp.float32),
            pltpu.VMEM((S_blk, 1), jnp.float32),
            pltpu.SemaphoreType.DMA((2,)),
            pltpu.SemaphoreType.DMA((2,)),
            pltpu.SemaphoreType.DMA((2,)),
        ],
        compiler_params=pltpu.CompilerParams(
            collective_id=0,
            vmem_limit_bytes=63 * 1024 * 1024,
        ),
    )(q, k, v)
